# baseline (speedup 1.0000x reference)
import math

import jax
import jax.numpy as jnp
from jax import lax
from jax.experimental import pallas as pl
from jax.experimental.pallas import tpu as pltpu
from jax.experimental.pallas import tpu_sc as plsc

VOCAB = 1000000
D = 64
BATCH = 4096
SEQ = 200
L = 16
GROUP = 128
B_BLK = 256
G_PER_UNIT = B_BLK // GROUP
BB_PER_S = BATCH // B_BLK
UNITS = SEQ * BB_PER_S
NW = 32
UNITS_PER_W = UNITS // NW
SCALE = math.sqrt(D)


def _emb_kernel(idx_hbm, tab_hbm, out_hbm, idx_v, rows_v, trans_v, sem):
    wid = lax.axis_index("s") * 2 + lax.axis_index("c")
    u0 = wid * UNITS_PER_W
    row_ids = [lax.iota(jnp.int32, L) + L * k for k in range(D // L)]

    def unit_body(i, carry):
        u = u0 + i
        s = u // BB_PER_S
        bb = u % BB_PER_S
        pltpu.sync_copy(idx_hbm.at[s, pl.ds(bb * G_PER_UNIT, G_PER_UNIT)], idx_v)
        copies = [
            pltpu.async_copy(
                tab_hbm.at[idx_v.at[g]],
                rows_v.at[pl.ds(g * GROUP, GROUP)],
                sem,
            )
            for g in range(G_PER_UNIT)
        ]
        for cp in copies:
            cp.wait()

        def tok_body(j, carry2):
            col = jnp.full((L,), j, jnp.int32)
            for k in range(D // L):
                val = rows_v[j, pl.ds(L * k, L)] * SCALE
                plsc.store_scatter(trans_v, [row_ids[k], col], val)
            return carry2

        lax.fori_loop(0, B_BLK, tok_body, 0, unroll=False)

        pltpu.sync_copy(trans_v, out_hbm.at[s, :, pl.ds(bb * B_BLK, B_BLK)])
        return carry

    lax.fori_loop(0, UNITS_PER_W, unit_body, 0, unroll=False)


@jax.jit
def kernel(token_indices, embedding_weight):
    idx3 = token_indices.T.reshape(SEQ, BATCH // GROUP, GROUP)
    mesh = plsc.VectorSubcoreMesh(core_axis_name="c", subcore_axis_name="s")
    out3 = pl.kernel(
        _emb_kernel,
        mesh=mesh,
        out_type=jax.ShapeDtypeStruct((SEQ, D, BATCH), jnp.float32),
        scratch_types=[
            pltpu.VMEM((G_PER_UNIT, GROUP), jnp.int32),
            pltpu.VMEM((B_BLK, D), jnp.float32),
            pltpu.VMEM((D, B_BLK), jnp.float32),
            pltpu.SemaphoreType.DMA,
        ],
        compiler_params=pltpu.CompilerParams(
            use_tc_tiling_on_sc=False, needs_layout_passes=False
        ),
    )(idx3, embedding_weight)
    return out3.transpose(2, 0, 1)


# speedup vs baseline: 1.4219x; 1.4219x over previous
import math

import jax
import jax.numpy as jnp
from jax import lax
from jax.experimental import pallas as pl
from jax.experimental.pallas import tpu as pltpu
from jax.experimental.pallas import tpu_sc as plsc

VOCAB = 1000000
D = 64
BATCH = 4096
SEQ = 200
L = 16
GROUP = 128
B_BLK = 256
G_PER_UNIT = B_BLK // GROUP
BB_PER_S = BATCH // B_BLK
UNITS = SEQ * BB_PER_S
NW = 32
UNITS_PER_W = UNITS // NW
SCALE = math.sqrt(D)


def _emb_kernel(idx_hbm, tab_hbm, out_hbm, idx_v, rows_v, trans_v, sem):
    wid = lax.axis_index("s") * 2 + lax.axis_index("c")
    u0 = wid * UNITS_PER_W
    row_ids = [lax.iota(jnp.int32, L) + L * k for k in range(D // L)]

    def unit_body(i, carry):
        u = u0 + i
        s = u // BB_PER_S
        bb = u % BB_PER_S
        pltpu.sync_copy(idx_hbm.at[s, pl.ds(bb * G_PER_UNIT, G_PER_UNIT)], idx_v)
        copies = [
            pltpu.async_copy(
                tab_hbm.at[idx_v.at[g]],
                rows_v.at[pl.ds(g * GROUP, GROUP)],
                sem,
            )
            for g in range(G_PER_UNIT)
        ]
        for cp in copies:
            cp.wait()

        def tok_body(j, carry2):
            col = jnp.full((L,), j, jnp.int32)
            for k in range(D // L):
                val = rows_v[j, pl.ds(L * k, L)] * SCALE
                plsc.store_scatter(trans_v, [row_ids[k], col], val)
            return carry2

        lax.fori_loop(0, B_BLK, tok_body, 0, unroll=4)

        pltpu.sync_copy(
            trans_v.at[:, pl.ds(0, B_BLK)],
            out_hbm.at[s, :, pl.ds(bb * B_BLK, B_BLK)],
        )
        return carry

    lax.fori_loop(0, UNITS_PER_W, unit_body, 0, unroll=False)


@jax.jit
def kernel(token_indices, embedding_weight):
    idx3 = token_indices.T.reshape(SEQ, BATCH // GROUP, GROUP)
    mesh = plsc.VectorSubcoreMesh(core_axis_name="c", subcore_axis_name="s")
    out3 = pl.kernel(
        _emb_kernel,
        mesh=mesh,
        out_type=jax.ShapeDtypeStruct((SEQ, D, BATCH), jnp.float32),
        scratch_types=[
            pltpu.VMEM((G_PER_UNIT, GROUP), jnp.int32),
            pltpu.VMEM((B_BLK, D), jnp.float32),
            pltpu.VMEM((D, B_BLK + 1), jnp.float32),
            pltpu.SemaphoreType.DMA,
        ],
        compiler_params=pltpu.CompilerParams(
            use_tc_tiling_on_sc=False, needs_layout_passes=False
        ),
    )(idx3, embedding_weight)
    return out3.transpose(2, 0, 1)


# speedup vs baseline: 1.6633x; 1.1698x over previous
import math

import jax
import jax.numpy as jnp
from jax import lax
from jax.experimental import pallas as pl
from jax.experimental.pallas import tpu as pltpu
from jax.experimental.pallas import tpu_sc as plsc

VOCAB = 1000000
D = 64
BATCH = 4096
SEQ = 200
L = 16
GROUP = 128
B_BLK = 256
G_PER_UNIT = B_BLK // GROUP
BB_PER_S = BATCH // B_BLK
UNITS = SEQ * BB_PER_S
NW = 32
UNITS_PER_W = UNITS // NW
SCALE = math.sqrt(D)


def _emb_kernel(idx_hbm, tab_hbm, out_hbm,
                idx_all, rows0, rows1, t0, t1,
                gsem0, gsem1, ssem0, ssem1):
    wid = lax.axis_index("s") * 2 + lax.axis_index("c")
    u0 = wid * UNITS_PER_W
    rows = (rows0, rows1)
    trans = (t0, t1)
    gsem = (gsem0, gsem1)
    ssem = (ssem0, ssem1)
    row_ids = [lax.iota(jnp.int32, L) + L * k for k in range(D // L)]

    pltpu.sync_copy(
        idx_hbm.at[pl.ds(u0 * G_PER_UNIT, UNITS_PER_W * G_PER_UNIT)], idx_all
    )

    def gather_copies(g, b):
        return [
            pltpu.async_copy(
                tab_hbm.at[idx_all.at[g * G_PER_UNIT + k]],
                rows[b].at[pl.ds(k * GROUP, GROUP)],
                gsem[b],
            )
            for k in range(G_PER_UNIT)
        ]

    def out_slice(g):
        u = u0 + g
        return out_hbm.at[u // BB_PER_S, :, pl.ds((u % BB_PER_S) * B_BLK, B_BLK)]

    def store_copy(g, b):
        return pltpu.async_copy(trans[b].at[:, pl.ds(0, B_BLK)], out_slice(g), ssem[b])

    gather_copies(0, 0)

    def pair_body(i, carry):
        for b in (0, 1):
            g = 2 * i + b
            for k in range(G_PER_UNIT):
                pltpu.make_async_copy(
                    tab_hbm.at[idx_all.at[k]],
                    rows[b].at[pl.ds(k * GROUP, GROUP)],
                    gsem[b],
                ).wait()

            @pl.when(g + 1 < UNITS_PER_W)
            def _():
                gather_copies(g + 1, 1 - b)

            @pl.when(g >= 2)
            def _():
                pltpu.make_async_copy(
                    trans[b].at[:, pl.ds(0, B_BLK)], out_slice(g), ssem[b]
                ).wait()

            def tok_body(j, carry2):
                col = jnp.full((L,), j, jnp.int32)
                for k in range(D // L):
                    val = rows[b][j, pl.ds(L * k, L)] * SCALE
                    plsc.store_scatter(trans[b], [row_ids[k], col], val)
                return carry2

            lax.fori_loop(0, B_BLK, tok_body, 0, unroll=4)

            store_copy(g, b)
        return carry

    lax.fori_loop(0, UNITS_PER_W // 2, pair_body, 0, unroll=False)

    for b in (0, 1):
        pltpu.make_async_copy(
            trans[b].at[:, pl.ds(0, B_BLK)],
            out_slice(UNITS_PER_W - 2 + b),
            ssem[b],
        ).wait()


@jax.jit
def kernel(token_indices, embedding_weight):
    idx2 = token_indices.T.reshape(UNITS * G_PER_UNIT, GROUP)
    mesh = plsc.VectorSubcoreMesh(core_axis_name="c", subcore_axis_name="s")
    out3 = pl.kernel(
        _emb_kernel,
        mesh=mesh,
        out_type=jax.ShapeDtypeStruct((SEQ, D, BATCH), jnp.float32),
        scratch_types=[
            pltpu.VMEM((UNITS_PER_W * G_PER_UNIT, GROUP), jnp.int32),
            pltpu.VMEM((B_BLK, D), jnp.float32),
            pltpu.VMEM((B_BLK, D), jnp.float32),
            pltpu.VMEM((D, B_BLK + 1), jnp.float32),
            pltpu.VMEM((D, B_BLK + 1), jnp.float32),
            pltpu.SemaphoreType.DMA,
            pltpu.SemaphoreType.DMA,
            pltpu.SemaphoreType.DMA,
            pltpu.SemaphoreType.DMA,
        ],
        compiler_params=pltpu.CompilerParams(
            use_tc_tiling_on_sc=False, needs_layout_passes=False
        ),
    )(idx2, embedding_weight)
    return out3.transpose(2, 0, 1)
